# baseline (speedup 1.0000x reference)
import functools

import jax
import jax.numpy as jnp
from jax import lax
from jax.experimental import pallas as pl
from jax.experimental.pallas import tpu as pltpu
from jax.experimental.pallas import tpu_sc as plsc

B, L, NF = 16384, 50, 5
VOCAB, DIM = 100000, 32

NC, NS, LANES = 2, 16, 16
NW = NC * NS
B_PER_W = B // NW

C = 256
G_IDX = 128
N_GROUPS = C // G_IDX
N_CHUNKS = 2 * L


def _body(xt_hbm, tab_hbm, out_hbm,
          xv_a, xv_b, rows_a, rows_b, outv_a, outv_b,
          sem_xa, sem_xb, sem_ga, sem_gb, sem_oa, sem_ob):
    wid = lax.axis_index("s") * NC + lax.axis_index("c")
    b_base = wid * B_PER_W

    def chunk_lb(chunk):
        return chunk >> 1, b_base + (chunk & 1) * C

    def xload(chunk, xv, sem):
        l, b0 = chunk_lb(chunk)
        for f in range(NF):
            pltpu.async_copy(xt_hbm.at[f, l, pl.ds(b0, C)], xv.at[f], sem)

    def xwait(xv, sem):
        pltpu.make_async_copy(
            xt_hbm.at[pl.ds(0, NF), 0, pl.ds(0, C)], xv, sem).wait()

    def fire(xv, rows, sem):
        for f in range(NF):
            for g in range(N_GROUPS):
                pltpu.async_copy(
                    tab_hbm.at[f].at[xv.at[f, pl.ds(g * G_IDX, G_IDX)]],
                    rows.at[f, pl.ds(g * G_IDX, G_IDX), :],
                    sem,
                )

    def gwait(rows, sem):
        pltpu.make_async_copy(
            tab_hbm.at[pl.ds(0, NF), pl.ds(0, C), :], rows, sem).wait()

    def reduce(rows, outv):
        @plsc.parallel_loop(0, C, unroll=4)
        def red_body(c):
            lo = rows[0, c, pl.ds(0, LANES)]
            hi = rows[0, c, pl.ds(LANES, LANES)]
            for t in range(1, NF):
                lo = lo + rows[t, c, pl.ds(0, LANES)]
                hi = hi + rows[t, c, pl.ds(LANES, LANES)]
            outv[c, pl.ds(0, LANES)] = lo
            outv[c, pl.ds(LANES, LANES)] = hi

    def owrite(chunk, outv, sem):
        l, b0 = chunk_lb(chunk)
        pltpu.async_copy(outv, out_hbm.at[pl.ds(b0, C), l, :], sem)

    def owait(outv, sem):
        pltpu.make_async_copy(outv, out_hbm.at[pl.ds(0, C), 0, :], sem).wait()

    xload(0, xv_a, sem_xa)
    xwait(xv_a, sem_xa)
    fire(xv_a, rows_a, sem_ga)
    xload(1, xv_b, sem_xb)

    def loop(kk, _):
        c0 = 2 * kk
        xwait(xv_b, sem_xb)
        fire(xv_b, rows_b, sem_gb)

        gwait(rows_a, sem_ga)

        @pl.when(kk > 0)
        def _():
            owait(outv_a, sem_oa)

        reduce(rows_a, outv_a)
        owrite(c0, outv_a, sem_oa)

        @pl.when(c0 + 2 < N_CHUNKS)
        def _():
            xload(c0 + 2, xv_a, sem_xa)
            xwait(xv_a, sem_xa)
            fire(xv_a, rows_a, sem_ga)
            xload(c0 + 3, xv_b, sem_xb)

        gwait(rows_b, sem_gb)

        @pl.when(kk > 0)
        def _():
            owait(outv_b, sem_ob)

        reduce(rows_b, outv_b)
        owrite(c0 + 1, outv_b, sem_ob)
        return _

    lax.fori_loop(0, N_CHUNKS // 2, loop, None)
    owait(outv_a, sem_oa)
    owait(outv_b, sem_ob)


@jax.jit
def _run(xt, tables):
    mesh = plsc.VectorSubcoreMesh(core_axis_name="c", subcore_axis_name="s")
    return pl.kernel(
        _body,
        mesh=mesh,
        compiler_params=pltpu.CompilerParams(use_tc_tiling_on_sc=False),
        out_type=jax.ShapeDtypeStruct((B, L, DIM), jnp.float32),
        scratch_types=[
            pltpu.VMEM((NF, C), jnp.int32),
            pltpu.VMEM((NF, C), jnp.int32),
            pltpu.VMEM((NF, C, DIM), jnp.float32),
            pltpu.VMEM((NF, C, DIM), jnp.float32),
            pltpu.VMEM((C, DIM), jnp.float32),
            pltpu.VMEM((C, DIM), jnp.float32),
            pltpu.SemaphoreType.DMA,
            pltpu.SemaphoreType.DMA,
            pltpu.SemaphoreType.DMA,
            pltpu.SemaphoreType.DMA,
            pltpu.SemaphoreType.DMA,
            pltpu.SemaphoreType.DMA,
        ],
    )(xt, tables)


def kernel(x, tables):
    xt = jnp.transpose(x, (2, 1, 0))
    return _run(xt, tables)


# speedup vs baseline: 1.1125x; 1.1125x over previous
import functools

import jax
import jax.numpy as jnp
from jax import lax
from jax.experimental import pallas as pl
from jax.experimental.pallas import tpu as pltpu
from jax.experimental.pallas import tpu_sc as plsc

B, L, NF = 16384, 50, 5
VOCAB, DIM = 100000, 32

NC, NS, LANES = 2, 16, 16
NW = NC * NS
B_PER_W = B // NW

C = 256
G_IDX = 128
N_GROUPS = C // G_IDX
N_CHUNKS = 2 * L


def _body(xt_hbm, tab_hbm, out_hbm,
          xv_a, xv_b, rows_a, rows_b, outv_a, outv_b,
          sem_xa, sem_xb, sem_ga, sem_gb, sem_oa, sem_ob):
    wid = lax.axis_index("s") * NC + lax.axis_index("c")
    b_base = wid * B_PER_W

    def chunk_lb(chunk):
        return chunk >> 1, b_base + (chunk & 1) * C

    def xload(chunk, xv, sem):
        l, b0 = chunk_lb(chunk)
        for f in range(NF):
            pltpu.async_copy(xt_hbm.at[f, l, pl.ds(b0, C)], xv.at[f], sem)

    def xwait(xv, sem):
        pltpu.make_async_copy(
            xt_hbm.at[pl.ds(0, NF), 0, pl.ds(0, C)], xv, sem).wait()

    def fire(xv, rows, sem):
        for f in range(NF):
            for g in range(N_GROUPS):
                pltpu.async_copy(
                    tab_hbm.at[f].at[xv.at[f, pl.ds(g * G_IDX, G_IDX)]],
                    rows.at[f, pl.ds(g * G_IDX, G_IDX), :],
                    sem,
                )

    def gwait(rows, sem):
        pltpu.make_async_copy(
            tab_hbm.at[pl.ds(0, NF), pl.ds(0, C), :], rows, sem).wait()

    iota = lax.iota(jnp.int32, LANES)
    pat_lo = (iota >> 3) * 2048 + (iota & 7) * 128
    pat_hi = pat_lo + 4096

    def reduce(rows, outv):
        @plsc.parallel_loop(0, C, unroll=4)
        def red_body(c):
            lo = rows[0, c, pl.ds(0, LANES)]
            hi = rows[0, c, pl.ds(LANES, LANES)]
            for t in range(1, NF):
                lo = lo + rows[t, c, pl.ds(0, LANES)]
                hi = hi + rows[t, c, pl.ds(LANES, LANES)]
            base = ((c >> 7) << 10) + (c & 127)
            plsc.store_scatter(outv, [pat_lo + base], lo)
            plsc.store_scatter(outv, [pat_hi + base], hi)

    def owrite(chunk, outv, sem):
        l, b0 = chunk_lb(chunk)
        tb0 = b0 >> 7
        for td in range(DIM // 8):
            pltpu.async_copy(
                outv.at[pl.ds(td * 2048, 2048)],
                out_hbm.at[l, pl.ds(td * (B * 8) + tb0 * 1024, 2048)],
                sem,
            )

    def owait(outv, sem):
        pltpu.make_async_copy(outv, out_hbm.at[0, pl.ds(0, C * DIM)], sem).wait()

    xload(0, xv_a, sem_xa)
    xwait(xv_a, sem_xa)
    fire(xv_a, rows_a, sem_ga)
    xload(1, xv_b, sem_xb)

    def loop(kk, _):
        c0 = 2 * kk
        xwait(xv_b, sem_xb)
        fire(xv_b, rows_b, sem_gb)

        gwait(rows_a, sem_ga)

        @pl.when(kk > 0)
        def _():
            owait(outv_a, sem_oa)

        reduce(rows_a, outv_a)
        owrite(c0, outv_a, sem_oa)

        @pl.when(c0 + 2 < N_CHUNKS)
        def _():
            xload(c0 + 2, xv_a, sem_xa)
            xwait(xv_a, sem_xa)
            fire(xv_a, rows_a, sem_ga)
            xload(c0 + 3, xv_b, sem_xb)

        gwait(rows_b, sem_gb)

        @pl.when(kk > 0)
        def _():
            owait(outv_b, sem_ob)

        reduce(rows_b, outv_b)
        owrite(c0 + 1, outv_b, sem_ob)
        return _

    lax.fori_loop(0, N_CHUNKS // 2, loop, None)
    owait(outv_a, sem_oa)
    owait(outv_b, sem_ob)


@jax.jit
def _run(xt, tables):
    mesh = plsc.VectorSubcoreMesh(core_axis_name="c", subcore_axis_name="s")
    return pl.kernel(
        _body,
        mesh=mesh,
        compiler_params=pltpu.CompilerParams(
            use_tc_tiling_on_sc=False, needs_layout_passes=False),
        out_type=jax.ShapeDtypeStruct((L, B * DIM), jnp.float32),
        scratch_types=[
            pltpu.VMEM((NF, C), jnp.int32),
            pltpu.VMEM((NF, C), jnp.int32),
            pltpu.VMEM((NF, C, DIM), jnp.float32),
            pltpu.VMEM((NF, C, DIM), jnp.float32),
            pltpu.VMEM((C * DIM,), jnp.float32),
            pltpu.VMEM((C * DIM,), jnp.float32),
            pltpu.SemaphoreType.DMA,
            pltpu.SemaphoreType.DMA,
            pltpu.SemaphoreType.DMA,
            pltpu.SemaphoreType.DMA,
            pltpu.SemaphoreType.DMA,
            pltpu.SemaphoreType.DMA,
        ],
    )(xt, tables)


def kernel(x, tables):
    xt = jnp.transpose(x, (2, 1, 0))
    out2d = _run(xt, tables)
    o5 = out2d.reshape(L, DIM // 8, B // 128, 8, 128)
    o5t = jnp.transpose(o5, (2, 4, 0, 1, 3))
    return o5t.reshape(B, L, DIM)


# speedup vs baseline: 1.8059x; 1.6233x over previous
import functools

import jax
import jax.numpy as jnp
from jax import lax
from jax.experimental import pallas as pl
from jax.experimental.pallas import tpu as pltpu
from jax.experimental.pallas import tpu_sc as plsc

B, L, NF = 16384, 50, 5
VOCAB, DIM = 100000, 32

NC, NS, LANES = 2, 16, 16
NW = NC * NS
B_PER_W = B // NW

C = 256
G_IDX = 128
N_GROUPS = C // G_IDX
N_CHUNKS = 2 * L


def _body(xt_hbm, tab_hbm, out_hbm,
          xv_a, xv_b, rows_a, rows_b, outv_a, outv_b,
          sem_xa, sem_xb, sem_ga, sem_gb, sem_oa, sem_ob):
    wid = lax.axis_index("s") * NC + lax.axis_index("c")
    b_base = wid * B_PER_W

    def chunk_lb(chunk):
        return chunk >> 1, b_base + (chunk & 1) * C

    def xload(chunk, xv, sem):
        l, b0 = chunk_lb(chunk)
        for f in range(NF):
            pltpu.async_copy(xt_hbm.at[f, l, pl.ds(b0, C)], xv.at[f], sem)

    def xwait(xv, sem):
        pltpu.make_async_copy(
            xt_hbm.at[pl.ds(0, NF), 0, pl.ds(0, C)], xv, sem).wait()

    def fire(xv, rows, sem):
        for f in range(NF):
            for g in range(N_GROUPS):
                pltpu.async_copy(
                    tab_hbm.at[f].at[xv.at[f, pl.ds(g * G_IDX, G_IDX)]],
                    rows.at[f, pl.ds(g * G_IDX, G_IDX), :],
                    sem,
                )

    def gwait(rows, sem):
        pltpu.make_async_copy(
            tab_hbm.at[pl.ds(0, NF), pl.ds(0, C), :], rows, sem).wait()

    iota = lax.iota(jnp.int32, LANES)
    td_lo = iota >> 3
    td_hi = td_lo + 2
    dr_v = iota & 7
    zeros = iota & 0

    def reduce(rows, outv):
        @plsc.parallel_loop(0, C, unroll=4)
        def red_body(c):
            lo = rows[0, c, pl.ds(0, LANES)]
            hi = rows[0, c, pl.ds(LANES, LANES)]
            for t in range(1, NF):
                lo = lo + rows[t, c, pl.ds(0, LANES)]
                hi = hi + rows[t, c, pl.ds(LANES, LANES)]
            tbb_v = zeros + (c >> 7)
            bc_v = zeros + (c & 127)
            plsc.store_scatter(outv, [td_lo, tbb_v, dr_v, bc_v], lo)
            plsc.store_scatter(outv, [td_hi, tbb_v, dr_v, bc_v], hi)

    def owrite(chunk, outv, sem):
        l, b0 = chunk_lb(chunk)
        tb0 = b0 >> 7
        for td in range(DIM // 8):
            for tbb in range(2):
                pltpu.async_copy(
                    outv.at[td, tbb, :, pl.ds(0, 128)],
                    out_hbm.at[l, td, tb0 + tbb, :, :],
                    sem,
                )

    def owait(outv, sem):
        for _ in range(DIM // 8):
            for tbb in range(2):
                pltpu.make_async_copy(
                    outv.at[0, 0, :, pl.ds(0, 128)],
                    out_hbm.at[0, 0, 0, :, :], sem).wait()

    xload(0, xv_a, sem_xa)
    xwait(xv_a, sem_xa)
    fire(xv_a, rows_a, sem_ga)
    xload(1, xv_b, sem_xb)

    def loop(kk, _):
        c0 = 2 * kk
        xwait(xv_b, sem_xb)
        fire(xv_b, rows_b, sem_gb)

        gwait(rows_a, sem_ga)

        @pl.when(kk > 0)
        def _():
            owait(outv_a, sem_oa)

        reduce(rows_a, outv_a)
        owrite(c0, outv_a, sem_oa)

        @pl.when(c0 + 2 < N_CHUNKS)
        def _():
            xload(c0 + 2, xv_a, sem_xa)
            xwait(xv_a, sem_xa)
            fire(xv_a, rows_a, sem_ga)
            xload(c0 + 3, xv_b, sem_xb)

        gwait(rows_b, sem_gb)

        @pl.when(kk > 0)
        def _():
            owait(outv_b, sem_ob)

        reduce(rows_b, outv_b)
        owrite(c0 + 1, outv_b, sem_ob)
        return _

    lax.fori_loop(0, N_CHUNKS // 2, loop, None)
    owait(outv_a, sem_oa)
    owait(outv_b, sem_ob)


@jax.jit
def _run(xt, tables):
    mesh = plsc.VectorSubcoreMesh(core_axis_name="c", subcore_axis_name="s")
    return pl.kernel(
        _body,
        mesh=mesh,
        compiler_params=pltpu.CompilerParams(
            use_tc_tiling_on_sc=False, needs_layout_passes=False),
        out_type=jax.ShapeDtypeStruct((L, DIM // 8, B // 128, 8, 128), jnp.float32),
        scratch_types=[
            pltpu.VMEM((NF, C), jnp.int32),
            pltpu.VMEM((NF, C), jnp.int32),
            pltpu.VMEM((NF, C, DIM), jnp.float32),
            pltpu.VMEM((NF, C, DIM), jnp.float32),
            pltpu.VMEM((DIM // 8, 3, 8, 129), jnp.float32),
            pltpu.VMEM((DIM // 8, 3, 8, 129), jnp.float32),
            pltpu.SemaphoreType.DMA,
            pltpu.SemaphoreType.DMA,
            pltpu.SemaphoreType.DMA,
            pltpu.SemaphoreType.DMA,
            pltpu.SemaphoreType.DMA,
            pltpu.SemaphoreType.DMA,
        ],
    )(xt, tables)


def kernel(x, tables):
    xt = jnp.transpose(x, (2, 1, 0))
    o5 = _run(xt, tables)
    o5t = jnp.transpose(o5, (2, 4, 0, 1, 3))
    return o5t.reshape(B, L, DIM)


# speedup vs baseline: 1.8239x; 1.0100x over previous
import functools

import jax
import jax.numpy as jnp
from jax import lax
from jax.experimental import pallas as pl
from jax.experimental.pallas import tpu as pltpu
from jax.experimental.pallas import tpu_sc as plsc

B, L, NF = 16384, 50, 5
VOCAB, DIM = 100000, 32

NC, NS, LANES = 2, 16, 16
NW = NC * NS
B_PER_W = B // NW

C = 256
G_IDX = 128
N_GROUPS = C // G_IDX
N_CHUNKS = 2 * L


def _body(xt_hbm, tab_hbm, out_hbm,
          xv_a, xv_b, rows_a, rows_b, outv_a, outv_b,
          sem_xa, sem_xb, sem_ga, sem_gb, sem_oa, sem_ob):
    wid = lax.axis_index("s") * NC + lax.axis_index("c")
    b_base = wid * B_PER_W

    def chunk_lb(chunk):
        return chunk >> 1, b_base + (chunk & 1) * C

    def xload(chunk, xv, sem):
        l, b0 = chunk_lb(chunk)
        for f in range(NF):
            pltpu.async_copy(xt_hbm.at[f, l, pl.ds(b0, C)], xv.at[f], sem)

    def xwait(xv, sem):
        pltpu.make_async_copy(
            xt_hbm.at[pl.ds(0, NF), 0, pl.ds(0, C)], xv, sem).wait()

    def fire(xv, rows, sem):
        for f in range(NF):
            for g in range(N_GROUPS):
                pltpu.async_copy(
                    tab_hbm.at[f].at[xv.at[f, pl.ds(g * G_IDX, G_IDX)]],
                    rows.at[f, pl.ds(g * G_IDX, G_IDX), :],
                    sem,
                )

    def gwait(rows, sem):
        pltpu.make_async_copy(
            tab_hbm.at[pl.ds(0, NF), pl.ds(0, C), :], rows, sem).wait()

    iota = lax.iota(jnp.int32, LANES)
    td_lo = iota >> 3
    td_hi = td_lo + 2
    dr_v = iota & 7
    zeros = iota & 0

    def reduce(rows, outv):
        @plsc.parallel_loop(0, C, unroll=8)
        def red_body(c):
            lo = rows[0, c, pl.ds(0, LANES)]
            hi = rows[0, c, pl.ds(LANES, LANES)]
            for t in range(1, NF):
                lo = lo + rows[t, c, pl.ds(0, LANES)]
                hi = hi + rows[t, c, pl.ds(LANES, LANES)]
            tbb_v = zeros + (c >> 7)
            bc_v = zeros + (c & 127)
            plsc.store_scatter(outv, [td_lo, tbb_v, dr_v, bc_v], lo)
            plsc.store_scatter(outv, [td_hi, tbb_v, dr_v, bc_v], hi)

    def owrite(chunk, outv, sem):
        l, b0 = chunk_lb(chunk)
        tb0 = b0 >> 7
        for td in range(DIM // 8):
            for tbb in range(2):
                pltpu.async_copy(
                    outv.at[td, tbb, :, pl.ds(0, 128)],
                    out_hbm.at[l, td, tb0 + tbb, :, :],
                    sem,
                )

    def owait(outv, sem):
        for _ in range(DIM // 8):
            for tbb in range(2):
                pltpu.make_async_copy(
                    outv.at[0, 0, :, pl.ds(0, 128)],
                    out_hbm.at[0, 0, 0, :, :], sem).wait()

    xload(0, xv_a, sem_xa)
    xwait(xv_a, sem_xa)
    fire(xv_a, rows_a, sem_ga)
    xload(1, xv_b, sem_xb)

    def loop(kk, _):
        c0 = 2 * kk
        xwait(xv_b, sem_xb)
        fire(xv_b, rows_b, sem_gb)

        gwait(rows_a, sem_ga)

        @pl.when(kk > 0)
        def _():
            owait(outv_a, sem_oa)

        reduce(rows_a, outv_a)
        owrite(c0, outv_a, sem_oa)

        @pl.when(c0 + 2 < N_CHUNKS)
        def _():
            xload(c0 + 2, xv_a, sem_xa)
            xwait(xv_a, sem_xa)
            fire(xv_a, rows_a, sem_ga)
            xload(c0 + 3, xv_b, sem_xb)

        gwait(rows_b, sem_gb)

        @pl.when(kk > 0)
        def _():
            owait(outv_b, sem_ob)

        reduce(rows_b, outv_b)
        owrite(c0 + 1, outv_b, sem_ob)
        return _

    lax.fori_loop(0, N_CHUNKS // 2, loop, None)
    owait(outv_a, sem_oa)
    owait(outv_b, sem_ob)


@jax.jit
def _run(xt, tables):
    mesh = plsc.VectorSubcoreMesh(core_axis_name="c", subcore_axis_name="s")
    return pl.kernel(
        _body,
        mesh=mesh,
        compiler_params=pltpu.CompilerParams(
            use_tc_tiling_on_sc=False, needs_layout_passes=False),
        out_type=jax.ShapeDtypeStruct((L, DIM // 8, B // 128, 8, 128), jnp.float32),
        scratch_types=[
            pltpu.VMEM((NF, C), jnp.int32),
            pltpu.VMEM((NF, C), jnp.int32),
            pltpu.VMEM((NF, C, DIM), jnp.float32),
            pltpu.VMEM((NF, C, DIM), jnp.float32),
            pltpu.VMEM((DIM // 8, 3, 8, 129), jnp.float32),
            pltpu.VMEM((DIM // 8, 3, 8, 129), jnp.float32),
            pltpu.SemaphoreType.DMA,
            pltpu.SemaphoreType.DMA,
            pltpu.SemaphoreType.DMA,
            pltpu.SemaphoreType.DMA,
            pltpu.SemaphoreType.DMA,
            pltpu.SemaphoreType.DMA,
        ],
    )(xt, tables)


def kernel(x, tables):
    xt = jnp.transpose(x, (2, 1, 0))
    o5 = _run(xt, tables)
    o5t = jnp.transpose(o5, (2, 4, 0, 1, 3))
    return o5t.reshape(B, L, DIM)
